# Initial kernel scaffold; baseline (speedup 1.0000x reference)
import jax
import jax.numpy as jnp
from jax.experimental import pallas as pl


def kernel(x, memory_keys, memory_values):
    raise NotImplementedError("write your pallas kernel here")


# speedup vs baseline: 2.6080x; 2.6080x over previous
import functools

import jax
import jax.numpy as jnp
from jax.experimental import pallas as pl
from jax.experimental.pallas import tpu as pltpu
from jax.experimental.pallas import tpu_sc as plsc

B = 1024
D = 256
N = 100000
K = 8
RES = 16
VD = RES * RES * RES
BK = B * K

CB = 2048
NB = (N + CB - 1) // CB
GW = 8

_NEG_INF = float("-inf")
_INT_MAX = 0x7FFFFFFF


def _topk_kernel(x_ref, k_ref, vals_ref, idx_ref, qn_ref):
    j = pl.program_id(0)

    @pl.when(j == 0)
    def _init():
        xq = x_ref[...]
        qn2 = jnp.sum(xq * xq, axis=1, keepdims=True)
        qn_ref[...] = xq / jnp.maximum(jnp.sqrt(qn2), 1e-12)
        vals_ref[...] = jnp.full((B, K), _NEG_INF, jnp.float32)
        idx_ref[...] = jnp.zeros((B, K), jnp.int32)

    kb = k_ref[...]
    kn2 = jnp.sum(kb * kb, axis=1, keepdims=True)
    kn = kb / jnp.maximum(jnp.sqrt(kn2), 1e-12)
    s = jax.lax.dot_general(
        qn_ref[...], kn, (((1,), (1,)), ((), ())),
        preferred_element_type=jnp.float32)
    col = jax.lax.broadcasted_iota(jnp.int32, (B, CB), 1) + j * CB
    s = jnp.where(col < N, s, _NEG_INF)

    rv = vals_ref[...]
    ri = idx_ref[...]
    lane = jax.lax.broadcasted_iota(jnp.int32, (B, K), 1)
    for _ in range(K):
        m = jnp.max(s, axis=1, keepdims=True)
        am = jnp.min(jnp.where(s == m, col, _INT_MAX), axis=1, keepdims=True)
        s = jnp.where(col == am, _NEG_INF, s)
        pos = jnp.sum((rv >= m).astype(jnp.int32), axis=1, keepdims=True)
        sv = jnp.concatenate(
            [jnp.full((B, 1), _NEG_INF, jnp.float32), rv[:, : K - 1]], axis=1)
        si = jnp.concatenate(
            [jnp.zeros((B, 1), jnp.int32), ri[:, : K - 1]], axis=1)
        mb = jnp.broadcast_to(m, (B, K))
        ab = jnp.broadcast_to(am, (B, K))
        rv = jnp.where(lane < pos, rv, jnp.where(lane == pos, mb, sv))
        ri = jnp.where(lane < pos, ri, jnp.where(lane == pos, ab, si))
    vals_ref[...] = rv
    idx_ref[...] = ri


def _topk_call(x, memory_keys):
    return pl.pallas_call(
        _topk_kernel,
        grid=(NB,),
        in_specs=[
            pl.BlockSpec((B, D), lambda j: (0, 0)),
            pl.BlockSpec((CB, D), lambda j: (j, 0)),
        ],
        out_specs=[
            pl.BlockSpec((B, K), lambda j: (0, 0)),
            pl.BlockSpec((B, K), lambda j: (0, 0)),
        ],
        out_shape=[
            jax.ShapeDtypeStruct((B, K), jnp.float32),
            jax.ShapeDtypeStruct((B, K), jnp.int32),
        ],
        scratch_shapes=[pltpu.VMEM((B, D), jnp.float32)],
    )(x, memory_keys)


_UNITS = 32
_PER = BK // _UNITS


def _gather_call(values2d, idx_flat):
    mesh = plsc.VectorSubcoreMesh(core_axis_name="c", subcore_axis_name="s")

    @functools.partial(
        pl.kernel,
        out_type=jax.ShapeDtypeStruct((BK, VD), jnp.float32),
        mesh=mesh,
        scratch_types=[
            pltpu.VMEM((_PER,), jnp.int32),
            pltpu.VMEM((GW, VD), jnp.float32),
        ],
    )
    def knl(x_hbm, i_hbm, o_hbm, idx_ref, stage_ref):
        c = jax.lax.axis_index("c")
        s = jax.lax.axis_index("s")
        base = (c * 16 + s) * _PER
        pltpu.sync_copy(i_hbm.at[pl.ds(base, _PER)], idx_ref)

        @pl.loop(0, _PER // GW)
        def _(w):
            win = idx_ref.at[pl.ds(w * GW, GW)]
            pltpu.sync_copy(x_hbm.at[win], stage_ref)
            pltpu.sync_copy(stage_ref, o_hbm.at[pl.ds(base + w * GW, GW)])

    return knl(values2d, idx_flat)


def kernel(x, memory_keys, memory_values):
    vals, idx = _topk_call(x, memory_keys)
    gathered = _gather_call(memory_values.reshape(N, VD), idx.reshape(BK))
    return gathered.reshape(B, K, RES, RES, RES), idx, vals


# speedup vs baseline: 3.4148x; 1.3094x over previous
import functools

import jax
import jax.numpy as jnp
from jax.experimental import pallas as pl
from jax.experimental.pallas import tpu as pltpu
from jax.experimental.pallas import tpu_sc as plsc

B = 1024
D = 256
N = 100000
K = 8
RES = 16
VD = RES * RES * RES
BK = B * K

CB = 2048
NB = (N + CB - 1) // CB
GW = 8

_NEG_INF = float("-inf")
_INT_MAX = 0x7FFFFFFF


def _topk_kernel(x_ref, k_ref, vals_ref, idx_ref, qn_ref):
    j = pl.program_id(0)

    @pl.when(j == 0)
    def _init():
        xq = x_ref[...]
        qn2 = jnp.sum(xq * xq, axis=1, keepdims=True)
        qn_ref[...] = xq / jnp.maximum(jnp.sqrt(qn2), 1e-12)
        vals_ref[...] = jnp.full((B, K), _NEG_INF, jnp.float32)
        idx_ref[...] = jnp.zeros((B, K), jnp.int32)

    kb = k_ref[...]
    kn2 = jnp.sum(kb * kb, axis=1, keepdims=True)
    kn = kb / jnp.maximum(jnp.sqrt(kn2), 1e-12)
    s = jax.lax.dot_general(
        qn_ref[...], kn, (((1,), (1,)), ((), ())),
        preferred_element_type=jnp.float32)
    col = jax.lax.broadcasted_iota(jnp.int32, (B, CB), 1) + j * CB
    s = jnp.where(col < N, s, _NEG_INF)

    rv = vals_ref[...]
    ri = idx_ref[...]
    lane = jax.lax.broadcasted_iota(jnp.int32, (B, K), 1)
    for _ in range(K):
        m = jnp.max(s, axis=1, keepdims=True)
        am = jnp.min(jnp.where(s == m, col, _INT_MAX), axis=1, keepdims=True)
        s = jnp.where(col == am, _NEG_INF, s)
        pos = jnp.sum((rv >= m).astype(jnp.int32), axis=1, keepdims=True)
        sv = jnp.concatenate(
            [jnp.full((B, 1), _NEG_INF, jnp.float32), rv[:, : K - 1]], axis=1)
        si = jnp.concatenate(
            [jnp.zeros((B, 1), jnp.int32), ri[:, : K - 1]], axis=1)
        mb = jnp.broadcast_to(m, (B, K))
        ab = jnp.broadcast_to(am, (B, K))
        rv = jnp.where(lane < pos, rv, jnp.where(lane == pos, mb, sv))
        ri = jnp.where(lane < pos, ri, jnp.where(lane == pos, ab, si))
    vals_ref[...] = rv
    idx_ref[...] = ri


def _topk_call(x, memory_keys):
    return pl.pallas_call(
        _topk_kernel,
        grid=(NB,),
        in_specs=[
            pl.BlockSpec((B, D), lambda j: (0, 0)),
            pl.BlockSpec((CB, D), lambda j: (j, 0)),
        ],
        out_specs=[
            pl.BlockSpec((B, K), lambda j: (0, 0)),
            pl.BlockSpec((B, K), lambda j: (0, 0)),
        ],
        out_shape=[
            jax.ShapeDtypeStruct((B, K), jnp.float32),
            jax.ShapeDtypeStruct((B, K), jnp.int32),
        ],
        scratch_shapes=[pltpu.VMEM((B, D), jnp.float32)],
    )(x, memory_keys)


_UNITS = 32
_PER = BK // _UNITS


def _gather_call(values2d, idx_flat):
    mesh = plsc.VectorSubcoreMesh(core_axis_name="c", subcore_axis_name="s")

    @functools.partial(
        pl.kernel,
        out_type=jax.ShapeDtypeStruct((B, K, VD), jnp.float32),
        mesh=mesh,
        scratch_types=[
            pltpu.VMEM((_PER,), jnp.int32),
            pltpu.VMEM((GW, VD), jnp.float32),
        ],
    )
    def knl(x_hbm, i_hbm, o_hbm, idx_ref, stage_ref):
        c = jax.lax.axis_index("c")
        s = jax.lax.axis_index("s")
        base = (c * 16 + s) * _PER
        pltpu.sync_copy(i_hbm.at[pl.ds(base, _PER)], idx_ref)

        @pl.loop(0, _PER // GW)
        def _(w):
            win = idx_ref.at[pl.ds(w * GW, GW)]
            pltpu.sync_copy(x_hbm.at[win], stage_ref)
            pltpu.sync_copy(stage_ref, o_hbm.at[(base + w * GW) // K])

    return knl(values2d, idx_flat)


def kernel(x, memory_keys, memory_values):
    vals, idx = _topk_call(x, memory_keys)
    gathered = _gather_call(memory_values.reshape(N, VD), idx.reshape(BK))
    return gathered.reshape(B, K, RES, RES, RES), idx, vals


# speedup vs baseline: 3.9402x; 1.1539x over previous
import functools

import jax
import jax.numpy as jnp
from jax.experimental import pallas as pl
from jax.experimental.pallas import tpu as pltpu
from jax.experimental.pallas import tpu_sc as plsc

B = 1024
D = 256
N = 100000
K = 8
RES = 16
VD = RES * RES * RES
BK = B * K

CB = 2048
NB = (N + CB - 1) // CB
_CH = 128
GW = 8

_NEG_INF = float("-inf")
_INT_MAX = 0x7FFFFFFF


def _topk_kernel(x_ref, k_ref, vals_ref, idx_ref, qn_ref):
    j = pl.program_id(0)

    @pl.when(j == 0)
    def _init():
        xq = x_ref[...]
        qn2 = jnp.sum(xq * xq, axis=1, keepdims=True)
        qn_ref[...] = xq / jnp.maximum(jnp.sqrt(qn2), 1e-12)
        vals_ref[...] = jnp.full((B, K), _NEG_INF, jnp.float32)
        idx_ref[...] = jnp.zeros((B, K), jnp.int32)

    kb = k_ref[...]
    kn2 = jnp.sum(kb * kb, axis=1, keepdims=True)
    kn = kb / jnp.maximum(jnp.sqrt(kn2), 1e-12)
    s = jax.lax.dot_general(
        qn_ref[...], kn, (((1,), (1,)), ((), ())),
        preferred_element_type=jnp.float32)
    col = jax.lax.broadcasted_iota(jnp.int32, (B, CB), 1) + j * CB
    s = jnp.where(col < N, s, _NEG_INF)

    m1 = m2 = m3 = jnp.full((B, _CH), _NEG_INF, jnp.float32)
    a1 = a2 = a3 = jnp.full((B, _CH), -1, jnp.int32)
    for c in range(CB // _CH):
        v = s[:, c * _CH:(c + 1) * _CH]
        vc = col[:, c * _CH:(c + 1) * _CH]
        g1 = v > m1
        g2 = v > m2
        g3 = v > m3
        nm1 = jnp.where(g1, v, m1)
        na1 = jnp.where(g1, vc, a1)
        nm2 = jnp.where(g1, m1, jnp.where(g2, v, m2))
        na2 = jnp.where(g1, a1, jnp.where(g2, vc, a2))
        nm3 = jnp.where(g1 | g2, m2, jnp.where(g3, v, m3))
        na3 = jnp.where(g1 | g2, a2, jnp.where(g3, vc, a3))
        m1, m2, m3, a1, a2, a3 = nm1, nm2, nm3, na1, na2, na3

    rv = vals_ref[...]
    ri = idx_ref[...]
    lane = jax.lax.broadcasted_iota(jnp.int32, (B, K), 1)
    for _ in range(K):
        m = jnp.max(m1, axis=1, keepdims=True)
        am = jnp.min(jnp.where(m1 == m, a1, _INT_MAX), axis=1, keepdims=True)
        won = a1 == am
        m1 = jnp.where(won, m2, m1)
        a1 = jnp.where(won, a2, a1)
        m2 = jnp.where(won, m3, m2)
        a2 = jnp.where(won, a3, a2)
        m3 = jnp.where(won, _NEG_INF, m3)
        pos = jnp.sum((rv >= m).astype(jnp.int32), axis=1, keepdims=True)
        sv = jnp.concatenate(
            [jnp.full((B, 1), _NEG_INF, jnp.float32), rv[:, : K - 1]], axis=1)
        si = jnp.concatenate(
            [jnp.zeros((B, 1), jnp.int32), ri[:, : K - 1]], axis=1)
        mb = jnp.broadcast_to(m, (B, K))
        ab = jnp.broadcast_to(am, (B, K))
        rv = jnp.where(lane < pos, rv, jnp.where(lane == pos, mb, sv))
        ri = jnp.where(lane < pos, ri, jnp.where(lane == pos, ab, si))
    vals_ref[...] = rv
    idx_ref[...] = ri


def _topk_call(x, memory_keys):
    return pl.pallas_call(
        _topk_kernel,
        grid=(NB,),
        in_specs=[
            pl.BlockSpec((B, D), lambda j: (0, 0)),
            pl.BlockSpec((CB, D), lambda j: (j, 0)),
        ],
        out_specs=[
            pl.BlockSpec((B, K), lambda j: (0, 0)),
            pl.BlockSpec((B, K), lambda j: (0, 0)),
        ],
        out_shape=[
            jax.ShapeDtypeStruct((B, K), jnp.float32),
            jax.ShapeDtypeStruct((B, K), jnp.int32),
        ],
        scratch_shapes=[pltpu.VMEM((B, D), jnp.float32)],
    )(x, memory_keys)


_UNITS = 32
_PER = BK // _UNITS


def _gather_call(values2d, idx_flat):
    mesh = plsc.VectorSubcoreMesh(core_axis_name="c", subcore_axis_name="s")

    @functools.partial(
        pl.kernel,
        out_type=jax.ShapeDtypeStruct((B, K, VD), jnp.float32),
        mesh=mesh,
        scratch_types=[
            pltpu.VMEM((_PER,), jnp.int32),
            pltpu.VMEM((GW, VD), jnp.float32),
        ],
    )
    def knl(x_hbm, i_hbm, o_hbm, idx_ref, stage_ref):
        c = jax.lax.axis_index("c")
        s = jax.lax.axis_index("s")
        base = (c * 16 + s) * _PER
        pltpu.sync_copy(i_hbm.at[pl.ds(base, _PER)], idx_ref)

        @pl.loop(0, _PER // GW)
        def _(w):
            win = idx_ref.at[pl.ds(w * GW, GW)]
            pltpu.sync_copy(x_hbm.at[win], stage_ref)
            pltpu.sync_copy(stage_ref, o_hbm.at[(base + w * GW) // K])

    return knl(values2d, idx_flat)


def kernel(x, memory_keys, memory_values):
    vals, idx = _topk_call(x, memory_keys)

    grp = (idx // CB) * _CH + (idx % _CH)
    gs = jnp.sort(grp, axis=1)
    bad = jnp.any(gs[:, 2:] == gs[:, : K - 2])

    def _exact_topk(_):
        qn = x / jnp.maximum(
            jnp.linalg.norm(x, ord=2, axis=1, keepdims=True), 1e-12)
        kn = memory_keys / jnp.maximum(
            jnp.linalg.norm(memory_keys, ord=2, axis=1, keepdims=True), 1e-12)
        tv, ti = jax.lax.top_k(jnp.matmul(qn, kn.T), K)
        return tv, ti

    vals, idx = jax.lax.cond(bad, _exact_topk, lambda _: (vals, idx), None)

    gathered = _gather_call(memory_values.reshape(N, VD), idx.reshape(BK))
    return gathered.reshape(B, K, RES, RES, RES), idx, vals


# speedup vs baseline: 4.0783x; 1.0350x over previous
import functools

import jax
import jax.numpy as jnp
from jax.experimental import pallas as pl
from jax.experimental.pallas import tpu as pltpu
from jax.experimental.pallas import tpu_sc as plsc

B = 1024
D = 256
N = 100000
K = 8
RES = 16
VD = RES * RES * RES
BK = B * K

CB = 2048
NB = (N + CB - 1) // CB
_CH = 128
GW = 8

_NEG_INF = float("-inf")
_INT_MAX = 0x7FFFFFFF


def _topk_kernel(x_ref, k_ref, vals_ref, idx_ref, qn_ref, ri_ref):
    j = pl.program_id(0)

    @pl.when(j == 0)
    def _init():
        xq = x_ref[...]
        qn2 = jnp.sum(xq * xq, axis=1, keepdims=True)
        qn_ref[...] = xq / jnp.maximum(jnp.sqrt(qn2), 1e-12)
        vals_ref[...] = jnp.full((B, K), _NEG_INF, jnp.float32)
        ri_ref[...] = jnp.zeros((B, K), jnp.float32)

    kb = k_ref[...]
    row = jax.lax.broadcasted_iota(jnp.int32, (CB, D), 0) + j * CB
    kb = jnp.where(row < N, kb, 0.0)
    kn2 = jnp.sum(kb * kb, axis=1, keepdims=True)
    kn = kb / jnp.maximum(jnp.sqrt(kn2), 1e-12)
    s = jax.lax.dot_general(
        qn_ref[...], kn, (((1,), (1,)), ((), ())),
        preferred_element_type=jnp.float32)

    lane_f = jax.lax.broadcasted_iota(
        jnp.int32, (B, _CH), 1).astype(jnp.float32)
    m1 = m2 = m3 = jnp.full((B, _CH), _NEG_INF, jnp.float32)
    a1 = a2 = a3 = jnp.full((B, _CH), -1.0, jnp.float32)
    boff = (j * CB).astype(jnp.float32)
    for c in range(CB // _CH):
        v = s[:, c * _CH:(c + 1) * _CH]
        vc = lane_f + (boff + jnp.float32(c * _CH))
        g1 = v > m1
        g2 = v > m2
        g3 = v > m3
        nm1 = jnp.where(g1, v, m1)
        na1 = jnp.where(g1, vc, a1)
        nm2 = jnp.where(g1, m1, jnp.where(g2, v, m2))
        na2 = jnp.where(g1, a1, jnp.where(g2, vc, a2))
        nm3 = jnp.where(g1 | g2, m2, jnp.where(g3, v, m3))
        na3 = jnp.where(g1 | g2, a2, jnp.where(g3, vc, a3))
        m1, m2, m3, a1, a2, a3 = nm1, nm2, nm3, na1, na2, na3

    rv = vals_ref[...]
    ri = ri_ref[...]
    for _ in range(K):
        m = jnp.max(m1, axis=1, keepdims=True)
        am = jnp.min(jnp.where(m1 == m, a1, jnp.float32(3.0e38)),
                     axis=1, keepdims=True)
        won = a1 == am
        m1 = jnp.where(won, m2, m1)
        a1 = jnp.where(won, a2, a1)
        m2 = jnp.where(won, m3, m2)
        a2 = jnp.where(won, a3, a2)
        m3 = jnp.where(won, _NEG_INF, m3)
        keep = rv >= m
        sv = jnp.concatenate(
            [jnp.full((B, 1), jnp.inf, jnp.float32), rv[:, : K - 1]], axis=1)
        ks = sv >= m
        si = jnp.concatenate(
            [jnp.zeros((B, 1), jnp.float32), ri[:, : K - 1]], axis=1)
        mb = jnp.broadcast_to(m, (B, K))
        ab = jnp.broadcast_to(am, (B, K))
        rv = jnp.where(keep, rv, jnp.where(ks, mb, sv))
        ri = jnp.where(keep, ri, jnp.where(ks, ab, si))
    vals_ref[...] = rv
    ri_ref[...] = ri

    @pl.when(j == NB - 1)
    def _fin():
        idx_ref[...] = ri_ref[...].astype(jnp.int32)


def _topk_call(x, memory_keys):
    return pl.pallas_call(
        _topk_kernel,
        grid=(NB,),
        in_specs=[
            pl.BlockSpec((B, D), lambda j: (0, 0)),
            pl.BlockSpec((CB, D), lambda j: (j, 0)),
        ],
        out_specs=[
            pl.BlockSpec((B, K), lambda j: (0, 0)),
            pl.BlockSpec((B, K), lambda j: (0, 0)),
        ],
        out_shape=[
            jax.ShapeDtypeStruct((B, K), jnp.float32),
            jax.ShapeDtypeStruct((B, K), jnp.int32),
        ],
        scratch_shapes=[
            pltpu.VMEM((B, D), jnp.float32),
            pltpu.VMEM((B, K), jnp.float32),
        ],
    )(x, memory_keys)


_UNITS = 32
_PER = BK // _UNITS


def _gather_call(values2d, idx_flat):
    mesh = plsc.VectorSubcoreMesh(core_axis_name="c", subcore_axis_name="s")

    @functools.partial(
        pl.kernel,
        out_type=jax.ShapeDtypeStruct((B, K, VD), jnp.float32),
        mesh=mesh,
        scratch_types=[
            pltpu.VMEM((_PER,), jnp.int32),
            pltpu.VMEM((GW, VD), jnp.float32),
        ],
    )
    def knl(x_hbm, i_hbm, o_hbm, idx_ref, stage_ref):
        c = jax.lax.axis_index("c")
        s = jax.lax.axis_index("s")
        base = (c * 16 + s) * _PER
        pltpu.sync_copy(i_hbm.at[pl.ds(base, _PER)], idx_ref)

        @pl.loop(0, _PER // GW)
        def _(w):
            win = idx_ref.at[pl.ds(w * GW, GW)]
            pltpu.sync_copy(x_hbm.at[win], stage_ref)
            pltpu.sync_copy(stage_ref, o_hbm.at[(base + w * GW) // K])

    return knl(values2d, idx_flat)


def kernel(x, memory_keys, memory_values):
    vals, idx = _topk_call(x, memory_keys)

    grp = (idx // CB) * _CH + (idx % _CH)
    gs = jnp.sort(grp, axis=1)
    bad = jnp.any(gs[:, 2:] == gs[:, : K - 2]) | jnp.any(idx >= N)

    def _exact_topk(_):
        qn = x / jnp.maximum(
            jnp.linalg.norm(x, ord=2, axis=1, keepdims=True), 1e-12)
        kn = memory_keys / jnp.maximum(
            jnp.linalg.norm(memory_keys, ord=2, axis=1, keepdims=True), 1e-12)
        tv, ti = jax.lax.top_k(jnp.matmul(qn, kn.T), K)
        return tv, ti

    vals, idx = jax.lax.cond(bad, _exact_topk, lambda _: (vals, idx), None)

    gathered = _gather_call(memory_values.reshape(N, VD), idx.reshape(BK))
    return gathered.reshape(B, K, RES, RES, RES), idx, vals
